# baseline (speedup 1.0000x reference)
import functools

import jax
import jax.numpy as jnp
from jax import lax
from jax.experimental import pallas as pl
from jax.experimental.pallas import tpu as pltpu
from jax.experimental.pallas import tpu_sc as plsc

N = 10000
E = 320000
D = 128

NC = 2
NS = 16
NW = NC * NS

CHUNK = 128
TOTCH = -(-E // CHUNK)
PAIR = 160
K0 = 80
K1 = PAIR - K0
KMAX = max(K0, K1)
TOT = NS * PAIR
EPAD = TOT * CHUNK

NPAD = 10112
ROWS_PER_TILE = NPAD // NS

_mesh = plsc.VectorSubcoreMesh(core_axis_name="c", subcore_axis_name="s")


def _deg_body(dst_hbm, ones_hbm, out_hbm, dst_v, ones_v, acc_sh):
    c = lax.axis_index("c")
    s = lax.axis_index("s")
    wid = c * NS + s
    DC = TOT // NW
    pltpu.sync_copy(dst_hbm.at[pl.ds(wid * DC, DC)], dst_v)
    pltpu.sync_copy(ones_hbm.at[pl.ds(0, CHUNK)], ones_v)
    pltpu.sync_copy(ones_hbm.at[pl.ds(CHUNK, ROWS_PER_TILE)],
                    acc_sh.at[pl.ds(s * ROWS_PER_TILE, ROWS_PER_TILE)])
    plsc.subcore_barrier()

    def body(j, carry):
        pltpu.sync_copy(ones_v, acc_sh.at[dst_v.at[j]], add=True)
        return carry

    lax.fori_loop(0, DC, body, 0)

    plsc.subcore_barrier()
    pltpu.sync_copy(acc_sh.at[pl.ds(s * ROWS_PER_TILE, ROWS_PER_TILE)],
                    out_hbm.at[c, pl.ds(s * ROWS_PER_TILE, ROWS_PER_TILE)])


def _sc_degree(dst_f, ones_pad):
    return pl.kernel(
        _deg_body,
        out_type=jax.ShapeDtypeStruct((NC, NPAD, 16), jnp.float32),
        mesh=_mesh,
        scratch_types=[
            pltpu.VMEM((TOT // NW, CHUNK), jnp.int32),
            pltpu.VMEM((CHUNK, 16), jnp.float32),
            pltpu.VMEM_SHARED((NPAD, 16), jnp.float32),
        ],
    )(dst_f, ones_pad)


def _agg_body(t_hbm, src_hbm, dst_hbm, zeros_hbm, out_hbm,
              src_v, dst_v, rows_v, acc_sh, sem):
    c = lax.axis_index("c")
    s = lax.axis_index("s")
    cnt = K0 + c * (K1 - K0)
    base = c * NS * K0 + s * cnt
    pltpu.sync_copy(src_hbm.at[pl.ds(base, KMAX)], src_v)
    pltpu.sync_copy(dst_hbm.at[pl.ds(base, KMAX)], dst_v)
    pltpu.sync_copy(zeros_hbm,
                    acc_sh.at[pl.ds(s * ROWS_PER_TILE, ROWS_PER_TILE)])
    plsc.subcore_barrier()

    def body(j, carry):
        cp = pltpu.async_copy(t_hbm.at[src_v.at[j]], rows_v, sem)
        cp.wait()
        pltpu.sync_copy(rows_v, acc_sh.at[dst_v.at[j]], add=True)
        return carry

    lax.fori_loop(0, cnt, body, 0)

    plsc.subcore_barrier()
    pltpu.sync_copy(acc_sh.at[pl.ds(s * ROWS_PER_TILE, ROWS_PER_TILE)],
                    out_hbm.at[c, pl.ds(s * ROWS_PER_TILE, ROWS_PER_TILE)])


def _sc_aggregate(t, src_f, dst_f, zeros_rows):
    return pl.kernel(
        _agg_body,
        out_type=jax.ShapeDtypeStruct((NC, NPAD, D), jnp.float32),
        mesh=_mesh,
        scratch_types=[
            pltpu.VMEM((KMAX, CHUNK), jnp.int32),
            pltpu.VMEM((KMAX, CHUNK), jnp.int32),
            pltpu.VMEM((CHUNK, D), jnp.float32),
            pltpu.VMEM_SHARED((NPAD, D), jnp.float32),
            pltpu.SemaphoreType.DMA,
        ],
    )(t, src_f, dst_f, zeros_rows)


_RB = 1000
_GRID = N // _RB


def _dinv_body(p_ref, o_ref):
    deg = p_ref[0, :, 0:1] + p_ref[1, :, 0:1] + 1.0
    o_ref[...] = jnp.broadcast_to(lax.rsqrt(deg), (_RB, D))


def _tc_dinv(deg_p):
    return pl.pallas_call(
        _dinv_body,
        grid=(_GRID,),
        in_specs=[pl.BlockSpec((NC, _RB, 16), lambda i: (0, i, 0))],
        out_specs=pl.BlockSpec((_RB, D), lambda i: (i, 0)),
        out_shape=jax.ShapeDtypeStruct((N, D), jnp.float32),
    )(deg_p)


def _matmul_body(x_ref, w_ref, dinv_ref, o_ref):
    h = jnp.dot(x_ref[...], w_ref[...], preferred_element_type=jnp.float32)
    o_ref[...] = dinv_ref[...] * h


def _tc_matmul(x, w, dinv_b):
    return pl.pallas_call(
        _matmul_body,
        grid=(_GRID,),
        in_specs=[
            pl.BlockSpec((_RB, D), lambda i: (i, 0)),
            pl.BlockSpec((D, D), lambda i: (0, 0)),
            pl.BlockSpec((_RB, D), lambda i: (i, 0)),
        ],
        out_specs=pl.BlockSpec((_RB, D), lambda i: (i, 0)),
        out_shape=jax.ShapeDtypeStruct((N, D), jnp.float32),
    )(x, w, dinv_b)


def _combine_body(relu, p0_ref, p1_ref, t_ref, dinv_ref, b_ref, o_ref):
    agg = p0_ref[0] + p1_ref[0]
    out = dinv_ref[...] * (agg + t_ref[...]) + b_ref[...]
    if relu:
        out = jnp.maximum(out, 0.0)
    o_ref[...] = out


def _tc_combine(p, t, dinv_b, b, relu):
    return pl.pallas_call(
        functools.partial(_combine_body, relu),
        grid=(_GRID,),
        in_specs=[
            pl.BlockSpec((1, _RB, D), lambda i: (0, i, 0)),
            pl.BlockSpec((1, _RB, D), lambda i: (1, i, 0)),
            pl.BlockSpec((_RB, D), lambda i: (i, 0)),
            pl.BlockSpec((_RB, D), lambda i: (i, 0)),
            pl.BlockSpec((1, D), lambda i: (0, 0)),
        ],
        out_specs=pl.BlockSpec((_RB, D), lambda i: (i, 0)),
        out_shape=jax.ShapeDtypeStruct((N, D), jnp.float32),
    )(p, p, t, dinv_b, b)


def kernel(x, edge_index, W1, b1, W2, b2, W3, b3):
    src = edge_index[0]
    dst = edge_index[1]
    pad = EPAD - E
    src_f = jnp.concatenate(
        [src, jnp.zeros((pad,), jnp.int32)]).reshape(TOT, CHUNK)
    dst_f = jnp.concatenate(
        [dst, jnp.full((pad,), N, jnp.int32)]).reshape(TOT, CHUNK)

    ones_pad = jnp.concatenate([
        jnp.ones((CHUNK, 16), jnp.float32),
        jnp.zeros((ROWS_PER_TILE, 16), jnp.float32)])
    zeros_rows = jnp.zeros((ROWS_PER_TILE, D), jnp.float32)

    deg_p = _sc_degree(dst_f, ones_pad)
    dinv_b = _tc_dinv(deg_p)

    h = x
    for (W, b, relu) in ((W1, b1, True), (W2, b2, True), (W3, b3, False)):
        t = _tc_matmul(h, W, dinv_b)
        p = _sc_aggregate(t, src_f, dst_f, zeros_rows)
        h = _tc_combine(p, t, dinv_b, b.reshape(1, D), relu)
    return h


# speedup vs baseline: 1.0059x; 1.0059x over previous
import functools

import jax
import jax.numpy as jnp
from jax import lax
from jax.experimental import pallas as pl
from jax.experimental.pallas import tpu as pltpu
from jax.experimental.pallas import tpu_sc as plsc

N = 10000
E = 320000
D = 128

NC = 2
NS = 16
NW = NC * NS

CHUNK = 128
TOTCH = -(-E // CHUNK)
PAIR = 160
K0 = 80
K1 = PAIR - K0
KMAX = max(K0, K1)
TOT = NS * PAIR
EPAD = TOT * CHUNK

NPAD = 10112
ROWS_PER_TILE = NPAD // NS

_mesh = plsc.VectorSubcoreMesh(core_axis_name="c", subcore_axis_name="s")


def _deg_body(dst_hbm, ones_hbm, out_hbm, dst_v, ones_v, acc_sh):
    c = lax.axis_index("c")
    s = lax.axis_index("s")
    wid = c * NS + s
    DC = TOT // NW
    pltpu.sync_copy(dst_hbm.at[pl.ds(wid * DC, DC)], dst_v)
    pltpu.sync_copy(ones_hbm.at[pl.ds(0, CHUNK)], ones_v)
    pltpu.sync_copy(ones_hbm.at[pl.ds(CHUNK, ROWS_PER_TILE)],
                    acc_sh.at[pl.ds(s * ROWS_PER_TILE, ROWS_PER_TILE)])
    plsc.subcore_barrier()

    def body(j, carry):
        pltpu.sync_copy(ones_v, acc_sh.at[dst_v.at[j]], add=True)
        return carry

    lax.fori_loop(0, DC, body, 0)

    plsc.subcore_barrier()
    pltpu.sync_copy(acc_sh.at[pl.ds(s * ROWS_PER_TILE, ROWS_PER_TILE)],
                    out_hbm.at[c, pl.ds(s * ROWS_PER_TILE, ROWS_PER_TILE)])


def _sc_degree(dst_f, ones_pad):
    return pl.kernel(
        _deg_body,
        out_type=jax.ShapeDtypeStruct((NC, NPAD, 16), jnp.float32),
        mesh=_mesh,
        scratch_types=[
            pltpu.VMEM((TOT // NW, CHUNK), jnp.int32),
            pltpu.VMEM((CHUNK, 16), jnp.float32),
            pltpu.VMEM_SHARED((NPAD, 16), jnp.float32),
        ],
    )(dst_f, ones_pad)


def _agg_body(t_hbm, src_hbm, dst_hbm, zeros_hbm, out_hbm,
              src_v, dst_v, rows_v, acc_sh, sem):
    c = lax.axis_index("c")
    s = lax.axis_index("s")
    pltpu.sync_copy(zeros_hbm,
                    acc_sh.at[pl.ds(s * ROWS_PER_TILE, ROWS_PER_TILE)])
    plsc.subcore_barrier()

    def body(j, carry):
        cp = pltpu.async_copy(t_hbm.at[src_v.at[j]], rows_v, sem)
        cp.wait()
        pltpu.sync_copy(rows_v, acc_sh.at[dst_v.at[j]], add=True)
        return carry

    def run(ci, k):
        base = ci * NS * K0 + s * k
        pltpu.sync_copy(src_hbm.at[pl.ds(base, k)], src_v.at[pl.ds(0, k)])
        pltpu.sync_copy(dst_hbm.at[pl.ds(base, k)], dst_v.at[pl.ds(0, k)])
        lax.fori_loop(0, k, body, 0)

    @pl.when(c == 0)
    def _():
        run(0, K0)

    @pl.when(c == 1)
    def _():
        run(1, K1)

    plsc.subcore_barrier()
    pltpu.sync_copy(acc_sh.at[pl.ds(s * ROWS_PER_TILE, ROWS_PER_TILE)],
                    out_hbm.at[c, pl.ds(s * ROWS_PER_TILE, ROWS_PER_TILE)])


def _sc_aggregate(t, src_f, dst_f, zeros_rows):
    return pl.kernel(
        _agg_body,
        out_type=jax.ShapeDtypeStruct((NC, NPAD, D), jnp.float32),
        mesh=_mesh,
        scratch_types=[
            pltpu.VMEM((KMAX, CHUNK), jnp.int32),
            pltpu.VMEM((KMAX, CHUNK), jnp.int32),
            pltpu.VMEM((CHUNK, D), jnp.float32),
            pltpu.VMEM_SHARED((NPAD, D), jnp.float32),
            pltpu.SemaphoreType.DMA,
        ],
    )(t, src_f, dst_f, zeros_rows)


_RB = 1000
_GRID = N // _RB


def _dinv_body(p_ref, o_ref):
    deg = p_ref[0, :, 0:1] + p_ref[1, :, 0:1] + 1.0
    o_ref[...] = jnp.broadcast_to(lax.rsqrt(deg), (_RB, D))


def _tc_dinv(deg_p):
    return pl.pallas_call(
        _dinv_body,
        grid=(_GRID,),
        in_specs=[pl.BlockSpec((NC, _RB, 16), lambda i: (0, i, 0))],
        out_specs=pl.BlockSpec((_RB, D), lambda i: (i, 0)),
        out_shape=jax.ShapeDtypeStruct((N, D), jnp.float32),
    )(deg_p)


def _matmul_body(x_ref, w_ref, dinv_ref, o_ref):
    h = jnp.dot(x_ref[...], w_ref[...], preferred_element_type=jnp.float32)
    o_ref[...] = dinv_ref[...] * h


def _tc_matmul(x, w, dinv_b):
    return pl.pallas_call(
        _matmul_body,
        grid=(_GRID,),
        in_specs=[
            pl.BlockSpec((_RB, D), lambda i: (i, 0)),
            pl.BlockSpec((D, D), lambda i: (0, 0)),
            pl.BlockSpec((_RB, D), lambda i: (i, 0)),
        ],
        out_specs=pl.BlockSpec((_RB, D), lambda i: (i, 0)),
        out_shape=jax.ShapeDtypeStruct((N, D), jnp.float32),
    )(x, w, dinv_b)


def _combine_body(relu, p0_ref, p1_ref, t_ref, dinv_ref, b_ref, o_ref):
    agg = p0_ref[0] + p1_ref[0]
    out = dinv_ref[...] * (agg + t_ref[...]) + b_ref[...]
    if relu:
        out = jnp.maximum(out, 0.0)
    o_ref[...] = out


def _tc_combine(p, t, dinv_b, b, relu):
    return pl.pallas_call(
        functools.partial(_combine_body, relu),
        grid=(_GRID,),
        in_specs=[
            pl.BlockSpec((1, _RB, D), lambda i: (0, i, 0)),
            pl.BlockSpec((1, _RB, D), lambda i: (1, i, 0)),
            pl.BlockSpec((_RB, D), lambda i: (i, 0)),
            pl.BlockSpec((_RB, D), lambda i: (i, 0)),
            pl.BlockSpec((1, D), lambda i: (0, 0)),
        ],
        out_specs=pl.BlockSpec((_RB, D), lambda i: (i, 0)),
        out_shape=jax.ShapeDtypeStruct((N, D), jnp.float32),
    )(p, p, t, dinv_b, b)


def kernel(x, edge_index, W1, b1, W2, b2, W3, b3):
    src = edge_index[0]
    dst = edge_index[1]
    pad = EPAD - E
    src_f = jnp.concatenate(
        [src, jnp.zeros((pad,), jnp.int32)]).reshape(TOT, CHUNK)
    dst_f = jnp.concatenate(
        [dst, jnp.full((pad,), N, jnp.int32)]).reshape(TOT, CHUNK)

    ones_pad = jnp.concatenate([
        jnp.ones((CHUNK, 16), jnp.float32),
        jnp.zeros((ROWS_PER_TILE, 16), jnp.float32)])
    zeros_rows = jnp.zeros((ROWS_PER_TILE, D), jnp.float32)

    deg_p = _sc_degree(dst_f, ones_pad)
    dinv_b = _tc_dinv(deg_p)

    h = x
    for (W, b, relu) in ((W1, b1, True), (W2, b2, True), (W3, b3, False)):
        t = _tc_matmul(h, W, dinv_b)
        p = _sc_aggregate(t, src_f, dst_f, zeros_rows)
        h = _tc_combine(p, t, dinv_b, b.reshape(1, D), relu)
    return h


# speedup vs baseline: 1.0080x; 1.0021x over previous
import functools

import jax
import jax.numpy as jnp
from jax import lax
from jax.experimental import pallas as pl
from jax.experimental.pallas import tpu as pltpu
from jax.experimental.pallas import tpu_sc as plsc

N = 10000
E = 320000
D = 128

NC = 2
NS = 16
NW = NC * NS

CHUNK = 128
PAIR = 160
K0 = 56
K1 = PAIR - K0
KMAX = max(K0, K1)
TOT = NS * PAIR
EPAD = TOT * CHUNK

NPAD = 10112
ROWS_PER_TILE = NPAD // NS

_mesh = plsc.VectorSubcoreMesh(core_axis_name="c", subcore_axis_name="s")


def _deg_body(dst_hbm, ones_hbm, out_hbm, dst_v, ones_v, acc_sh):
    c = lax.axis_index("c")
    s = lax.axis_index("s")
    wid = c * NS + s
    pltpu.sync_copy(dst_hbm.at[wid], dst_v)
    pltpu.sync_copy(ones_hbm.at[pl.ds(0, CHUNK)], ones_v)
    pltpu.sync_copy(ones_hbm.at[pl.ds(CHUNK, ROWS_PER_TILE)],
                    acc_sh.at[pl.ds(s * ROWS_PER_TILE, ROWS_PER_TILE)])
    plsc.subcore_barrier()

    def body(j, carry):
        pltpu.sync_copy(ones_v, acc_sh.at[dst_v.at[j]], add=True)
        return carry

    lax.fori_loop(0, TOT // NW, body, 0)

    plsc.subcore_barrier()
    pltpu.sync_copy(acc_sh.at[pl.ds(s * ROWS_PER_TILE, ROWS_PER_TILE)],
                    out_hbm.at[c, pl.ds(s * ROWS_PER_TILE, ROWS_PER_TILE)])


def _sc_degree(dst_b, ones_pad):
    return pl.kernel(
        _deg_body,
        out_type=jax.ShapeDtypeStruct((NC, NPAD, 16), jnp.float32),
        mesh=_mesh,
        scratch_types=[
            pltpu.VMEM((TOT // NW, CHUNK), jnp.int32),
            pltpu.VMEM((CHUNK, 16), jnp.float32),
            pltpu.VMEM_SHARED((NPAD, 16), jnp.float32),
        ],
    )(dst_b, ones_pad)


def _agg_body(t_hbm, src_hbm, dst_hbm, zeros_hbm, out_hbm,
              src_v, dst_v, rows_v, acc_sh, sem):
    c = lax.axis_index("c")
    s = lax.axis_index("s")
    wid = c * NS + s
    pltpu.sync_copy(src_hbm.at[wid], src_v)
    pltpu.sync_copy(dst_hbm.at[wid], dst_v)
    pltpu.sync_copy(zeros_hbm,
                    acc_sh.at[pl.ds(s * ROWS_PER_TILE, ROWS_PER_TILE)])
    plsc.subcore_barrier()

    def body(j, carry):
        cp = pltpu.async_copy(t_hbm.at[src_v.at[j]], rows_v, sem)
        cp.wait()
        pltpu.sync_copy(rows_v, acc_sh.at[dst_v.at[j]], add=True)
        return carry

    @pl.when(c == 0)
    def _():
        lax.fori_loop(0, K0, body, 0)

    @pl.when(c == 1)
    def _():
        lax.fori_loop(0, K1, body, 0)

    plsc.subcore_barrier()
    pltpu.sync_copy(acc_sh.at[pl.ds(s * ROWS_PER_TILE, ROWS_PER_TILE)],
                    out_hbm.at[c, pl.ds(s * ROWS_PER_TILE, ROWS_PER_TILE)])


def _sc_aggregate(t, src_w, dst_w, zeros_rows):
    return pl.kernel(
        _agg_body,
        out_type=jax.ShapeDtypeStruct((NC, NPAD, D), jnp.float32),
        mesh=_mesh,
        scratch_types=[
            pltpu.VMEM((KMAX, CHUNK), jnp.int32),
            pltpu.VMEM((KMAX, CHUNK), jnp.int32),
            pltpu.VMEM((CHUNK, D), jnp.float32),
            pltpu.VMEM_SHARED((NPAD, D), jnp.float32),
            pltpu.SemaphoreType.DMA,
        ],
    )(t, src_w, dst_w, zeros_rows)


_RB = 1000
_GRID = N // _RB


def _dinv_body(p_ref, o_ref):
    deg = p_ref[0, :, 0:1] + p_ref[1, :, 0:1] + 1.0
    o_ref[...] = jnp.broadcast_to(lax.rsqrt(deg), (_RB, D))


def _tc_dinv(deg_p):
    return pl.pallas_call(
        _dinv_body,
        grid=(_GRID,),
        in_specs=[pl.BlockSpec((NC, _RB, 16), lambda i: (0, i, 0))],
        out_specs=pl.BlockSpec((_RB, D), lambda i: (i, 0)),
        out_shape=jax.ShapeDtypeStruct((N, D), jnp.float32),
    )(deg_p)


def _matmul_body(x_ref, w_ref, dinv_ref, o_ref):
    h = jnp.dot(x_ref[...], w_ref[...], preferred_element_type=jnp.float32)
    o_ref[...] = dinv_ref[...] * h


def _tc_matmul(x, w, dinv_b):
    return pl.pallas_call(
        _matmul_body,
        grid=(_GRID,),
        in_specs=[
            pl.BlockSpec((_RB, D), lambda i: (i, 0)),
            pl.BlockSpec((D, D), lambda i: (0, 0)),
            pl.BlockSpec((_RB, D), lambda i: (i, 0)),
        ],
        out_specs=pl.BlockSpec((_RB, D), lambda i: (i, 0)),
        out_shape=jax.ShapeDtypeStruct((N, D), jnp.float32),
    )(x, w, dinv_b)


def _combine_body(relu, p0_ref, p1_ref, t_ref, dinv_ref, b_ref, o_ref):
    agg = p0_ref[0] + p1_ref[0]
    out = dinv_ref[...] * (agg + t_ref[...]) + b_ref[...]
    if relu:
        out = jnp.maximum(out, 0.0)
    o_ref[...] = out


def _tc_combine(p, t, dinv_b, b, relu):
    return pl.pallas_call(
        functools.partial(_combine_body, relu),
        grid=(_GRID,),
        in_specs=[
            pl.BlockSpec((1, _RB, D), lambda i: (0, i, 0)),
            pl.BlockSpec((1, _RB, D), lambda i: (1, i, 0)),
            pl.BlockSpec((_RB, D), lambda i: (i, 0)),
            pl.BlockSpec((_RB, D), lambda i: (i, 0)),
            pl.BlockSpec((1, D), lambda i: (0, 0)),
        ],
        out_specs=pl.BlockSpec((_RB, D), lambda i: (i, 0)),
        out_shape=jax.ShapeDtypeStruct((N, D), jnp.float32),
    )(p, p, t, dinv_b, b)


def kernel(x, edge_index, W1, b1, W2, b2, W3, b3):
    src = edge_index[0]
    dst = edge_index[1]
    pad = EPAD - E
    src_f = jnp.concatenate(
        [src, jnp.zeros((pad,), jnp.int32)]).reshape(TOT, CHUNK)
    dst_f = jnp.concatenate(
        [dst, jnp.full((pad,), N, jnp.int32)]).reshape(TOT, CHUNK)

    w0 = src_f[:NS * K0].reshape(NS, K0, CHUNK)
    w1 = src_f[NS * K0:].reshape(NS, K1, CHUNK)
    d0 = dst_f[:NS * K0].reshape(NS, K0, CHUNK)
    d1 = dst_f[NS * K0:].reshape(NS, K1, CHUNK)

    def padw(a, k, fill):
        if k == KMAX:
            return a
        return jnp.concatenate(
            [a, jnp.full((NS, KMAX - k, CHUNK), fill, jnp.int32)], axis=1)

    src_w = jnp.concatenate([padw(w0, K0, 0), padw(w1, K1, 0)], axis=0)
    dst_w = jnp.concatenate([padw(d0, K0, N), padw(d1, K1, N)], axis=0)

    ones_pad = jnp.concatenate([
        jnp.ones((CHUNK, 16), jnp.float32),
        jnp.zeros((ROWS_PER_TILE, 16), jnp.float32)])
    zeros_rows = jnp.zeros((ROWS_PER_TILE, D), jnp.float32)

    deg_p = _sc_degree(dst_f.reshape(NW, TOT // NW, CHUNK), ones_pad)
    dinv_b = _tc_dinv(deg_p)

    h = x
    for (W, b, relu) in ((W1, b1, True), (W2, b2, True), (W3, b3, False)):
        t = _tc_matmul(h, W, dinv_b)
        p = _sc_aggregate(t, src_w, dst_w, zeros_rows)
        h = _tc_combine(p, t, dinv_b, b.reshape(1, D), relu)
    return h


# speedup vs baseline: 1.0553x; 1.0469x over previous
import functools

import jax
import jax.numpy as jnp
from jax import lax
from jax.experimental import pallas as pl
from jax.experimental.pallas import tpu as pltpu
from jax.experimental.pallas import tpu_sc as plsc

N = 10000
E = 320000
D = 128

NC = 2
NS = 16
NW = NC * NS

CHUNK = 128
PAIR = 160
K0 = 80
K1 = PAIR - K0
KMAX = max(K0, K1)
TOT = NS * PAIR
EPAD = TOT * CHUNK

NPAD = 10112
ROWS_PER_TILE = NPAD // NS

_mesh = plsc.VectorSubcoreMesh(core_axis_name="c", subcore_axis_name="s")


def _deg_body(dst_hbm, ones_hbm, out_hbm, dst_v, ones_v, acc_sh):
    c = lax.axis_index("c")
    s = lax.axis_index("s")
    wid = c * NS + s
    pltpu.sync_copy(dst_hbm.at[wid], dst_v)
    pltpu.sync_copy(ones_hbm.at[pl.ds(0, CHUNK)], ones_v)
    pltpu.sync_copy(ones_hbm.at[pl.ds(CHUNK, ROWS_PER_TILE)],
                    acc_sh.at[pl.ds(s * ROWS_PER_TILE, ROWS_PER_TILE)])
    plsc.subcore_barrier()

    def body(j, carry):
        pltpu.sync_copy(ones_v, acc_sh.at[dst_v.at[j]], add=True)
        return carry

    lax.fori_loop(0, TOT // NW, body, 0)

    plsc.subcore_barrier()
    pltpu.sync_copy(acc_sh.at[pl.ds(s * ROWS_PER_TILE, ROWS_PER_TILE)],
                    out_hbm.at[c, pl.ds(s * ROWS_PER_TILE, ROWS_PER_TILE)])


def _sc_degree(dst_b, ones_pad):
    return pl.kernel(
        _deg_body,
        out_type=jax.ShapeDtypeStruct((NC, NPAD, 16), jnp.float32),
        mesh=_mesh,
        scratch_types=[
            pltpu.VMEM((TOT // NW, CHUNK), jnp.int32),
            pltpu.VMEM((CHUNK, 16), jnp.float32),
            pltpu.VMEM_SHARED((NPAD, 16), jnp.float32),
        ],
    )(dst_b, ones_pad)


def _agg_body(t_hbm, src_hbm, dst_hbm, zeros_hbm, out_hbm,
              src_v, dst_v, rows_v, acc_sh, sem):
    c = lax.axis_index("c")
    s = lax.axis_index("s")
    wid = c * NS + s
    pltpu.sync_copy(src_hbm.at[wid], src_v)
    pltpu.sync_copy(dst_hbm.at[wid], dst_v)
    pltpu.sync_copy(zeros_hbm,
                    acc_sh.at[pl.ds(s * ROWS_PER_TILE, ROWS_PER_TILE)])
    plsc.subcore_barrier()

    def body(j, carry):
        cp = pltpu.async_copy(t_hbm.at[src_v.at[j]], rows_v, sem)
        cp.wait()
        pltpu.sync_copy(rows_v, acc_sh.at[dst_v.at[j]], add=True)
        return carry

    @pl.when(c == 0)
    def _():
        lax.fori_loop(0, K0, body, 0)

    @pl.when(c == 1)
    def _():
        lax.fori_loop(0, K1, body, 0)

    plsc.subcore_barrier()
    pltpu.sync_copy(acc_sh.at[pl.ds(s * ROWS_PER_TILE, ROWS_PER_TILE)],
                    out_hbm.at[c, pl.ds(s * ROWS_PER_TILE, ROWS_PER_TILE)])


def _sc_aggregate(t, src_w, dst_w, zeros_rows):
    return pl.kernel(
        _agg_body,
        out_type=jax.ShapeDtypeStruct((NC, NPAD, D), jnp.float32),
        mesh=_mesh,
        scratch_types=[
            pltpu.VMEM((KMAX, CHUNK), jnp.int32),
            pltpu.VMEM((KMAX, CHUNK), jnp.int32),
            pltpu.VMEM((CHUNK, D), jnp.float32),
            pltpu.VMEM_SHARED((NPAD, D), jnp.float32),
            pltpu.SemaphoreType.DMA,
        ],
    )(t, src_w, dst_w, zeros_rows)


_RB = 1000
_GRID = N // _RB


def _dinv_body(p_ref, o_ref):
    deg = p_ref[0, :, 0:1] + p_ref[1, :, 0:1] + 1.0
    o_ref[...] = jnp.broadcast_to(lax.rsqrt(deg), (_RB, D))


def _tc_dinv(deg_p):
    return pl.pallas_call(
        _dinv_body,
        grid=(_GRID,),
        in_specs=[pl.BlockSpec((NC, _RB, 16), lambda i: (0, i, 0))],
        out_specs=pl.BlockSpec((_RB, D), lambda i: (i, 0)),
        out_shape=jax.ShapeDtypeStruct((N, D), jnp.float32),
    )(deg_p)


def _matmul_body(x_ref, w_ref, dinv_ref, o_ref):
    h = jnp.dot(x_ref[...], w_ref[...], preferred_element_type=jnp.float32)
    o_ref[...] = dinv_ref[...] * h


def _tc_matmul(x, w, dinv_b):
    return pl.pallas_call(
        _matmul_body,
        grid=(_GRID,),
        in_specs=[
            pl.BlockSpec((_RB, D), lambda i: (i, 0)),
            pl.BlockSpec((D, D), lambda i: (0, 0)),
            pl.BlockSpec((_RB, D), lambda i: (i, 0)),
        ],
        out_specs=pl.BlockSpec((_RB, D), lambda i: (i, 0)),
        out_shape=jax.ShapeDtypeStruct((N, D), jnp.float32),
    )(x, w, dinv_b)


def _combine_body(relu, p0_ref, p1_ref, t_ref, dinv_ref, b_ref, o_ref):
    agg = p0_ref[0] + p1_ref[0]
    out = dinv_ref[...] * (agg + t_ref[...]) + b_ref[...]
    if relu:
        out = jnp.maximum(out, 0.0)
    o_ref[...] = out


def _tc_combine(p, t, dinv_b, b, relu):
    return pl.pallas_call(
        functools.partial(_combine_body, relu),
        grid=(_GRID,),
        in_specs=[
            pl.BlockSpec((1, _RB, D), lambda i: (0, i, 0)),
            pl.BlockSpec((1, _RB, D), lambda i: (1, i, 0)),
            pl.BlockSpec((_RB, D), lambda i: (i, 0)),
            pl.BlockSpec((_RB, D), lambda i: (i, 0)),
            pl.BlockSpec((1, D), lambda i: (0, 0)),
        ],
        out_specs=pl.BlockSpec((_RB, D), lambda i: (i, 0)),
        out_shape=jax.ShapeDtypeStruct((N, D), jnp.float32),
    )(p, p, t, dinv_b, b)


def kernel(x, edge_index, W1, b1, W2, b2, W3, b3):
    src = edge_index[0]
    dst = edge_index[1]
    pad = EPAD - E
    dummy_dst = N + (jnp.arange(pad, dtype=jnp.int32) % (NPAD - N))
    src_f = jnp.concatenate(
        [src, jnp.zeros((pad,), jnp.int32)]).reshape(TOT, CHUNK)
    dst_f = jnp.concatenate([dst, dummy_dst]).reshape(TOT, CHUNK)

    w0 = src_f[:NS * K0].reshape(NS, K0, CHUNK)
    w1 = src_f[NS * K0:].reshape(NS, K1, CHUNK)
    d0 = dst_f[:NS * K0].reshape(NS, K0, CHUNK)
    d1 = dst_f[NS * K0:].reshape(NS, K1, CHUNK)

    def padw(a, k, fill):
        if k == KMAX:
            return a
        return jnp.concatenate(
            [a, jnp.full((NS, KMAX - k, CHUNK), fill, jnp.int32)], axis=1)

    src_w = jnp.concatenate([padw(w0, K0, 0), padw(w1, K1, 0)], axis=0)
    dst_w = jnp.concatenate([padw(d0, K0, N), padw(d1, K1, N)], axis=0)

    ones_pad = jnp.concatenate([
        jnp.ones((CHUNK, 16), jnp.float32),
        jnp.zeros((ROWS_PER_TILE, 16), jnp.float32)])
    zeros_rows = jnp.zeros((ROWS_PER_TILE, D), jnp.float32)

    deg_p = _sc_degree(dst_f.reshape(NW, TOT // NW, CHUNK), ones_pad)
    dinv_b = _tc_dinv(deg_p)

    h = x
    for (W, b, relu) in ((W1, b1, True), (W2, b2, True), (W3, b3, False)):
        t = _tc_matmul(h, W, dinv_b)
        p = _sc_aggregate(t, src_w, dst_w, zeros_rows)
        h = _tc_combine(p, t, dinv_b, b.reshape(1, D), relu)
    return h


# speedup vs baseline: 1.6332x; 1.5477x over previous
import functools

import jax
import jax.numpy as jnp
from jax import lax
from jax.experimental import pallas as pl
from jax.experimental.pallas import tpu as pltpu
from jax.experimental.pallas import tpu_sc as plsc

N = 10000
E = 320000
D = 128

NC = 2
NS = 16
NW = NC * NS

CHUNK = 128
EPW = ((E // NW + CHUNK - 1) // CHUNK) * CHUNK
NCHUNK = EPW // CHUNK
EPAD = EPW * NW

NPAD = 10112
ROWS_PER_TILE = NPAD // NS

_mesh = plsc.VectorSubcoreMesh(core_axis_name="c", subcore_axis_name="s")


def _deg_body(dst_hbm, ones_hbm, out_hbm, dst_v, ones_v, acc_sh):
    c = lax.axis_index("c")
    s = lax.axis_index("s")
    wid = c * NS + s
    pltpu.sync_copy(dst_hbm.at[wid], dst_v)
    pltpu.sync_copy(ones_hbm.at[pl.ds(0, CHUNK)], ones_v)
    pltpu.sync_copy(ones_hbm.at[pl.ds(128, ROWS_PER_TILE)],
                    acc_sh.at[pl.ds(s * ROWS_PER_TILE, ROWS_PER_TILE)])
    plsc.subcore_barrier()

    def body(j, carry):
        pltpu.sync_copy(ones_v, acc_sh.at[dst_v.at[j]], add=True)
        return carry

    lax.fori_loop(0, NCHUNK, body, 0)
    plsc.subcore_barrier()
    pltpu.sync_copy(acc_sh.at[pl.ds(s * ROWS_PER_TILE, ROWS_PER_TILE)],
                    out_hbm.at[c, pl.ds(s * ROWS_PER_TILE, ROWS_PER_TILE)])


def _sc_degree(dst_w, ones_pad):
    return pl.kernel(
        _deg_body,
        out_type=jax.ShapeDtypeStruct((NC, NPAD, 16), jnp.float32),
        mesh=_mesh,
        scratch_types=[
            pltpu.VMEM((NCHUNK, CHUNK), jnp.int32),
            pltpu.VMEM((CHUNK, 16), jnp.float32),
            pltpu.VMEM_SHARED((NPAD, 16), jnp.float32),
        ],
    )(dst_w, ones_pad)


def _agg_body(t_hbm, src_hbm, dst_hbm, zeros_hbm, out_hbm,
              src_v, dst_v, rows_v, acc_sh, sem):
    c = lax.axis_index("c")
    s = lax.axis_index("s")
    wid = c * NS + s
    pltpu.sync_copy(src_hbm.at[wid], src_v)
    pltpu.sync_copy(dst_hbm.at[wid], dst_v)
    pltpu.sync_copy(zeros_hbm.at[pl.ds(0, ROWS_PER_TILE)],
                    acc_sh.at[pl.ds(s * ROWS_PER_TILE, ROWS_PER_TILE)])
    plsc.subcore_barrier()

    def body(j, carry):
        cp = pltpu.async_copy(t_hbm.at[src_v.at[j]], rows_v, sem)
        cp.wait()
        pltpu.sync_copy(rows_v, acc_sh.at[dst_v.at[j]], add=True)
        return carry

    lax.fori_loop(0, NCHUNK, body, 0)
    plsc.subcore_barrier()
    pltpu.sync_copy(acc_sh.at[pl.ds(s * ROWS_PER_TILE, ROWS_PER_TILE)],
                    out_hbm.at[c, pl.ds(s * ROWS_PER_TILE, ROWS_PER_TILE)])


def _sc_aggregate(t, src_w, dst_w, zeros_rows):
    return pl.kernel(
        _agg_body,
        out_type=jax.ShapeDtypeStruct((NC, NPAD, D), jnp.float32),
        mesh=_mesh,
        scratch_types=[
            pltpu.VMEM((NCHUNK, CHUNK), jnp.int32),
            pltpu.VMEM((NCHUNK, CHUNK), jnp.int32),
            pltpu.VMEM((CHUNK, D), jnp.float32),
            pltpu.VMEM_SHARED((NPAD, D), jnp.float32),
            pltpu.SemaphoreType.DMA,
        ],
    )(t, src_w, dst_w, zeros_rows)


_RB = 1000
_GRID = N // _RB


def _dinv_body(p_ref, o_ref):
    deg = p_ref[0, :, 0:1] + p_ref[1, :, 0:1] + 1.0
    o_ref[...] = jnp.broadcast_to(lax.rsqrt(deg), (_RB, D))


def _tc_dinv(deg_p):
    return pl.pallas_call(
        _dinv_body,
        grid=(_GRID,),
        in_specs=[pl.BlockSpec((NC, _RB, 16), lambda i: (0, i, 0))],
        out_specs=pl.BlockSpec((_RB, D), lambda i: (i, 0)),
        out_shape=jax.ShapeDtypeStruct((N, D), jnp.float32),
    )(deg_p)


def _matmul_body(x_ref, w_ref, dinv_ref, o_ref):
    h = jnp.dot(x_ref[...], w_ref[...], preferred_element_type=jnp.float32)
    o_ref[...] = dinv_ref[...] * h


def _tc_matmul(x, w, dinv_b):
    return pl.pallas_call(
        _matmul_body,
        grid=(_GRID,),
        in_specs=[
            pl.BlockSpec((_RB, D), lambda i: (i, 0)),
            pl.BlockSpec((D, D), lambda i: (0, 0)),
            pl.BlockSpec((_RB, D), lambda i: (i, 0)),
        ],
        out_specs=pl.BlockSpec((_RB, D), lambda i: (i, 0)),
        out_shape=jax.ShapeDtypeStruct((N, D), jnp.float32),
    )(x, w, dinv_b)


def _combine_body(relu, p0_ref, p1_ref, t_ref, dinv_ref, b_ref, o_ref):
    s = p0_ref[0] + p1_ref[0] + t_ref[...]
    out = dinv_ref[...] * s + b_ref[...]
    if relu:
        out = jnp.maximum(out, 0.0)
    o_ref[...] = out


def _tc_combine(p, t, dinv_b, b, relu):
    return pl.pallas_call(
        functools.partial(_combine_body, relu),
        grid=(_GRID,),
        in_specs=[
            pl.BlockSpec((1, _RB, D), lambda i: (0, i, 0)),
            pl.BlockSpec((1, _RB, D), lambda i: (1, i, 0)),
            pl.BlockSpec((_RB, D), lambda i: (i, 0)),
            pl.BlockSpec((_RB, D), lambda i: (i, 0)),
            pl.BlockSpec((1, D), lambda i: (0, 0)),
        ],
        out_specs=pl.BlockSpec((_RB, D), lambda i: (i, 0)),
        out_shape=jax.ShapeDtypeStruct((N, D), jnp.float32),
    )(p, p, t, dinv_b, b)


def kernel(x, edge_index, W1, b1, W2, b2, W3, b3):
    src = edge_index[0]
    dst = edge_index[1]
    pad = EPAD - E
    src_w = jnp.concatenate(
        [src, jnp.zeros((pad,), jnp.int32)]).reshape(NW, NCHUNK, CHUNK)
    dst_w = jnp.concatenate(
        [dst, jnp.full((pad,), N, jnp.int32)]).reshape(NW, NCHUNK, CHUNK)

    ones_pad = jnp.concatenate([
        jnp.ones((CHUNK, 16), jnp.float32),
        jnp.zeros((ROWS_PER_TILE, 16), jnp.float32)])
    zeros_rows = jnp.zeros((ROWS_PER_TILE, D), jnp.float32)

    deg_p = _sc_degree(dst_w, ones_pad)
    dinv_b = _tc_dinv(deg_p)

    h = x
    for (W, b, relu) in ((W1, b1, True), (W2, b2, True), (W3, b3, False)):
        t = _tc_matmul(h, W, dinv_b)
        p = _sc_aggregate(t, src_w, dst_w, zeros_rows)
        h = _tc_combine(p, t, dinv_b, b.reshape(1, D), relu)
    return h


# speedup vs baseline: 1.6811x; 1.0293x over previous
import functools

import jax
import jax.numpy as jnp
from jax import lax
from jax.experimental import pallas as pl
from jax.experimental.pallas import tpu as pltpu
from jax.experimental.pallas import tpu_sc as plsc

N = 10000
E = 320000
D = 128

NC = 2
NS = 16
NW = NC * NS

CHUNK = 128
EPW = ((E // NW + CHUNK - 1) // CHUNK) * CHUNK
NCHUNK = EPW // CHUNK
EPAD = EPW * NW

NPAD = 10112
ROWS_PER_TILE = NPAD // NS

_mesh = plsc.VectorSubcoreMesh(core_axis_name="c", subcore_axis_name="s")


def _deg_body(dst_hbm, ones_hbm, out_hbm, dst_v, ones_v, acc_sh):
    c = lax.axis_index("c")
    s = lax.axis_index("s")
    wid = c * NS + s
    pltpu.sync_copy(dst_hbm.at[wid], dst_v)
    pltpu.sync_copy(ones_hbm.at[pl.ds(0, CHUNK)], ones_v)
    pltpu.sync_copy(ones_hbm.at[pl.ds(128, ROWS_PER_TILE)],
                    acc_sh.at[pl.ds(s * ROWS_PER_TILE, ROWS_PER_TILE)])
    plsc.subcore_barrier()

    def body(j, carry):
        pltpu.sync_copy(ones_v, acc_sh.at[dst_v.at[j]], add=True)
        return carry

    lax.fori_loop(0, NCHUNK, body, 0)
    plsc.subcore_barrier()
    pltpu.sync_copy(acc_sh.at[pl.ds(s * ROWS_PER_TILE, ROWS_PER_TILE)],
                    out_hbm.at[c, pl.ds(s * ROWS_PER_TILE, ROWS_PER_TILE)])


def _sc_degree(dst_w, ones_pad):
    return pl.kernel(
        _deg_body,
        out_type=jax.ShapeDtypeStruct((NC, NPAD, 16), jnp.float32),
        mesh=_mesh,
        scratch_types=[
            pltpu.VMEM((NCHUNK, CHUNK), jnp.int32),
            pltpu.VMEM((CHUNK, 16), jnp.float32),
            pltpu.VMEM_SHARED((NPAD, 16), jnp.float32),
        ],
    )(dst_w, ones_pad)


def _agg_body(t_hbm, src_hbm, dst_hbm, zeros_hbm, out_hbm,
              src_v, dst_v, rows_v, acc_sh, sem):
    c = lax.axis_index("c")
    s = lax.axis_index("s")
    wid = c * NS + s
    pltpu.sync_copy(src_hbm.at[wid], src_v)
    pltpu.sync_copy(dst_hbm.at[wid], dst_v)
    pltpu.sync_copy(zeros_hbm.at[pl.ds(0, ROWS_PER_TILE)],
                    acc_sh.at[pl.ds(s * ROWS_PER_TILE, ROWS_PER_TILE)])
    plsc.subcore_barrier()

    def body(j, carry):
        cp = pltpu.async_copy(t_hbm.at[src_v.at[j]], rows_v, sem)
        cp.wait()
        pltpu.sync_copy(rows_v, acc_sh.at[dst_v.at[j]], add=True)
        return carry

    lax.fori_loop(0, NCHUNK, body, 0)
    plsc.subcore_barrier()
    pltpu.sync_copy(acc_sh.at[pl.ds(s * ROWS_PER_TILE, ROWS_PER_TILE)],
                    out_hbm.at[c, pl.ds(s * ROWS_PER_TILE, ROWS_PER_TILE)])


def _sc_aggregate(t, src_w, dst_w, zeros_rows):
    return pl.kernel(
        _agg_body,
        out_type=jax.ShapeDtypeStruct((NC, NPAD, D), jnp.float32),
        mesh=_mesh,
        scratch_types=[
            pltpu.VMEM((NCHUNK, CHUNK), jnp.int32),
            pltpu.VMEM((NCHUNK, CHUNK), jnp.int32),
            pltpu.VMEM((CHUNK, D), jnp.float32),
            pltpu.VMEM_SHARED((NPAD, D), jnp.float32),
            pltpu.SemaphoreType.DMA,
        ],
    )(t, src_w, dst_w, zeros_rows)


_RB = 1000
_GRID = N // _RB


def _dinv_body(p_ref, o_ref):
    deg = p_ref[0, :, 0:1] + p_ref[1, :, 0:1] + 1.0
    o_ref[...] = jnp.broadcast_to(lax.rsqrt(deg), (_RB, D))


def _tc_dinv(deg_p):
    return pl.pallas_call(
        _dinv_body,
        grid=(_GRID,),
        in_specs=[pl.BlockSpec((NC, _RB, 16), lambda i: (0, i, 0))],
        out_specs=pl.BlockSpec((_RB, D), lambda i: (i, 0)),
        out_shape=jax.ShapeDtypeStruct((N, D), jnp.float32),
    )(deg_p)


def _matmul_body(x_ref, w_ref, dinv_ref, o_ref):
    h = jnp.dot(x_ref[...], w_ref[...], preferred_element_type=jnp.float32)
    o_ref[...] = dinv_ref[...] * h


def _tc_matmul(x, w, dinv_b):
    return pl.pallas_call(
        _matmul_body,
        grid=(_GRID,),
        in_specs=[
            pl.BlockSpec((_RB, D), lambda i: (i, 0)),
            pl.BlockSpec((D, D), lambda i: (0, 0)),
            pl.BlockSpec((_RB, D), lambda i: (i, 0)),
        ],
        out_specs=pl.BlockSpec((_RB, D), lambda i: (i, 0)),
        out_shape=jax.ShapeDtypeStruct((N, D), jnp.float32),
    )(x, w, dinv_b)


def _combine_body(relu, p0_ref, p1_ref, t_ref, dinv_ref, b_ref, o_ref):
    s = p0_ref[0] + p1_ref[0] + t_ref[...]
    out = dinv_ref[...] * s + b_ref[...]
    if relu:
        out = jnp.maximum(out, 0.0)
    o_ref[...] = out


def _tc_combine(p, t, dinv_b, b, relu):
    return pl.pallas_call(
        functools.partial(_combine_body, relu),
        grid=(_GRID,),
        in_specs=[
            pl.BlockSpec((1, _RB, D), lambda i: (0, i, 0)),
            pl.BlockSpec((1, _RB, D), lambda i: (1, i, 0)),
            pl.BlockSpec((_RB, D), lambda i: (i, 0)),
            pl.BlockSpec((_RB, D), lambda i: (i, 0)),
            pl.BlockSpec((1, D), lambda i: (0, 0)),
        ],
        out_specs=pl.BlockSpec((_RB, D), lambda i: (i, 0)),
        out_shape=jax.ShapeDtypeStruct((N, D), jnp.float32),
    )(p, p, t, dinv_b, b)


def _fused_body(p0_ref, p1_ref, t_ref, dinv_ref, b_ref, w_ref, o_ref):
    s = p0_ref[0] + p1_ref[0] + t_ref[...]
    h = jnp.maximum(dinv_ref[...] * s + b_ref[...], 0.0)
    o_ref[...] = dinv_ref[...] * jnp.dot(
        h, w_ref[...], preferred_element_type=jnp.float32)


def _tc_combine_matmul(p, t, dinv_b, b, w):
    return pl.pallas_call(
        _fused_body,
        grid=(_GRID,),
        in_specs=[
            pl.BlockSpec((1, _RB, D), lambda i: (0, i, 0)),
            pl.BlockSpec((1, _RB, D), lambda i: (1, i, 0)),
            pl.BlockSpec((_RB, D), lambda i: (i, 0)),
            pl.BlockSpec((_RB, D), lambda i: (i, 0)),
            pl.BlockSpec((1, D), lambda i: (0, 0)),
            pl.BlockSpec((D, D), lambda i: (0, 0)),
        ],
        out_specs=pl.BlockSpec((_RB, D), lambda i: (i, 0)),
        out_shape=jax.ShapeDtypeStruct((N, D), jnp.float32),
    )(p, p, t, dinv_b, b, w)


def kernel(x, edge_index, W1, b1, W2, b2, W3, b3):
    src = edge_index[0]
    dst = edge_index[1]
    pad = EPAD - E
    src_w = jnp.concatenate(
        [src, jnp.zeros((pad,), jnp.int32)]).reshape(NW, NCHUNK, CHUNK)
    dst_w = jnp.concatenate(
        [dst, jnp.full((pad,), N, jnp.int32)]).reshape(NW, NCHUNK, CHUNK)

    ones_pad = jnp.concatenate([
        jnp.ones((CHUNK, 16), jnp.float32),
        jnp.zeros((ROWS_PER_TILE, 16), jnp.float32)])
    zeros_rows = jnp.zeros((ROWS_PER_TILE, D), jnp.float32)

    deg_p = _sc_degree(dst_w, ones_pad)
    dinv_b = _tc_dinv(deg_p)

    t = _tc_matmul(x, W1, dinv_b)
    p = _sc_aggregate(t, src_w, dst_w, zeros_rows)
    t = _tc_combine_matmul(p, t, dinv_b, b1.reshape(1, D), W2)
    p = _sc_aggregate(t, src_w, dst_w, zeros_rows)
    t = _tc_combine_matmul(p, t, dinv_b, b2.reshape(1, D), W3)
    p = _sc_aggregate(t, src_w, dst_w, zeros_rows)
    return _tc_combine(p, t, dinv_b, b3.reshape(1, D), False)
